# baseline (speedup 1.0000x reference)
import functools

import jax
import jax.numpy as jnp
from jax import lax
from jax.experimental import pallas as pl
from jax.experimental.pallas import tpu as pltpu
from jax.experimental.pallas import tpu_sc as plsc

_D = 1024
_ROWS = 4096
_BATCH = 4
_NW = 32
_ROWS_PER_W = _ROWS // _NW
_CHUNK = 8
_NCHUNK = _ROWS_PER_W // _CHUNK
_CELEM = _CHUNK * _D
_GRP = 16
_NGRP = _CELEM // (16 * _GRP)


def _body(x_hbm, sel_hbm, pos_hbm, out_hbm, idx_v, pe_v, xb_v, *sems):
    pes = sems[0:2]
    xl = sems[2:4]
    ss = sems[4:6]
    wid = lax.axis_index("s") * 2 + lax.axis_index("c")
    base = wid * _ROWS_PER_W
    n_grps = _NGRP + (wid >> 5)

    def row0(c):
        return base + c * _CHUNK

    def start_pe(c):
        p = c % 2
        pltpu.sync_copy(sel_hbm.at[pl.ds(row0(c), _CHUNK)], idx_v.at[p])
        return pltpu.async_copy(pos_hbm.at[idx_v.at[p]], pe_v.at[p], pes[p])

    def start_loads(c):
        q = c % 2
        return [pltpu.async_copy(
            x_hbm.at[b, pl.ds(row0(c) * _D, _CELEM)],
            xb_v.at[q, pl.ds(b * _CELEM, _CELEM)], xl[q])
            for b in range(_BATCH)]

    def start_stores(c):
        q = c % 2
        return [pltpu.async_copy(
            xb_v.at[q, pl.ds(b * _CELEM, _CELEM)],
            out_hbm.at[b, pl.ds(row0(c) * _D, _CELEM)], ss[q])
            for b in range(_BATCH)]

    pe_h = {0: start_pe(0)}
    ld_h = {0: start_loads(0)}
    st_h = {}
    for c in range(_NCHUNK):
        p = c % 2
        q = c % 2
        pe_h[c].wait()
        if c + 1 < _NCHUNK:
            pe_h[c + 1] = start_pe(c + 1)
        for h in ld_h[c]:
            h.wait()

        def add_grp(g, carry, q=q, p=p):
            xoff, r, jb = carry
            xoff = pl.multiple_of(xoff, 16)
            jb = pl.multiple_of(jb, 16)
            for u in range(_GRP):
                v = pe_v[p, r, pl.ds(jb + u * 16, 16)]
                for b in range(_BATCH):
                    plsc.addupdate(
                        xb_v.at[q, pl.ds(xoff + b * _CELEM + u * 16, 16)], v)
            last = jb == _D - _GRP * 16
            return (xoff + _GRP * 16,
                    r + jnp.where(last, 1, 0),
                    jnp.where(last, 0, jb + _GRP * 16))

        lax.fori_loop(0, n_grps, add_grp,
                      (jnp.int32(0), jnp.int32(0), jnp.int32(0)))
        st_h[c] = start_stores(c)
        if c + 1 < _NCHUNK:
            if c - 1 >= 0:
                for h in st_h[c - 1]:
                    h.wait()
            ld_h[c + 1] = start_loads(c + 1)
    for c in (_NCHUNK - 2, _NCHUNK - 1):
        for h in st_h[c]:
            h.wait()


def kernel(x, pos_table, hash_index):
    sel = hash_index[:64, :64].reshape(-1).astype(jnp.int32)
    pos2 = pos_table.reshape(pos_table.shape[1], _D)
    x2 = x.reshape(_BATCH, _ROWS * _D)
    mesh = plsc.VectorSubcoreMesh(core_axis_name="c", subcore_axis_name="s")
    run = functools.partial(
        pl.kernel,
        out_type=jax.ShapeDtypeStruct((_BATCH, _ROWS * _D), jnp.float32),
        mesh=mesh,
        compiler_params=pltpu.CompilerParams(use_tc_tiling_on_sc=False),
        scratch_types=[
            pltpu.VMEM((2, _CHUNK), jnp.int32),
            pltpu.VMEM((2, _CHUNK, _D), jnp.float32),
            pltpu.VMEM((2, _BATCH * _CELEM), jnp.float32),
        ] + [pltpu.SemaphoreType.DMA] * 6,
    )(_body)
    out = run(x2, sel, pos2)
    return out.reshape(_BATCH, _ROWS, _D)


# speedup vs baseline: 1.1045x; 1.1045x over previous
import functools

import jax
import jax.numpy as jnp
from jax import lax
from jax.experimental import pallas as pl
from jax.experimental.pallas import tpu as pltpu
from jax.experimental.pallas import tpu_sc as plsc

_D = 1024
_ROWS = 4096
_BATCH = 4
_NW = 32
_ROWS_PER_W = _ROWS // _NW
_CHUNK = 8
_NCHUNK = _ROWS_PER_W // _CHUNK
_CELEM = _CHUNK * _D
_GRP = 16
_NGRP = _CELEM // (16 * _GRP)


def _body(x_hbm, sel_hbm, pos_hbm, out_hbm, idx_v, pe_v, xb_v, *sems):
    pes = sems[0:2]
    xl = sems[2:4]
    ss = sems[4:6]
    wid = lax.axis_index("s") * 2 + lax.axis_index("c")
    base = wid * _ROWS_PER_W
    n_grps = _NGRP + (wid >> 5)

    def row0(c):
        return base + c * _CHUNK

    def start_pe(c):
        p = c % 2
        pltpu.sync_copy(sel_hbm.at[pl.ds(row0(c), _CHUNK)], idx_v.at[p])
        return pltpu.async_copy(pos_hbm.at[idx_v.at[p]], pe_v.at[p], pes[p])

    def start_loads(c):
        q = c % 2
        return [pltpu.async_copy(
            x_hbm.at[b, pl.ds(row0(c) * _D, _CELEM)],
            xb_v.at[q, pl.ds(b * _CELEM, _CELEM)], xl[q])
            for b in range(_BATCH)]

    def start_stores(c):
        q = c % 2
        return [pltpu.async_copy(
            xb_v.at[q, pl.ds(b * _CELEM, _CELEM)],
            out_hbm.at[b, pl.ds(row0(c) * _D, _CELEM)], ss[q])
            for b in range(_BATCH)]

    pe_h = {0: start_pe(0)}
    ld_h = {0: start_loads(0)}
    st_h = {}
    for c in range(_NCHUNK):
        p = c % 2
        q = c % 2
        pe_h[c].wait()
        if c + 1 < _NCHUNK:
            pe_h[c + 1] = start_pe(c + 1)
        for h in ld_h[c]:
            h.wait()

        def add_grp(g, carry, q=q, p=p):
            xoff, r, jb = carry
            xoff = pl.multiple_of(xoff, 16)
            jb = pl.multiple_of(jb, 16)
            for u in range(_GRP):
                v = pe_v[p, r, pl.ds(jb + u * 16, 16)]
                for b in range(_BATCH):
                    plsc.addupdate(
                        xb_v.at[q, pl.ds(xoff + b * _CELEM + u * 16, 16)], v)
            last = jb == _D - _GRP * 16
            return (xoff + _GRP * 16,
                    r + jnp.where(last, 1, 0),
                    jnp.where(last, 0, jb + _GRP * 16))

        lax.fori_loop(0, n_grps, add_grp,
                      (jnp.int32(0), jnp.int32(0), jnp.int32(0)))
        st_h[c] = start_stores(c)
        if c + 1 < _NCHUNK:
            if c - 1 >= 0:
                for h in st_h[c - 1]:
                    h.wait()
            ld_h[c + 1] = start_loads(c + 1)
    for c in (_NCHUNK - 2, _NCHUNK - 1):
        for h in st_h[c]:
            h.wait()


def kernel(x, pos_table, hash_index):
    sel = hash_index[:64, :64].reshape(-1).astype(jnp.int32)
    pos2 = pos_table.reshape(pos_table.shape[1], _D)
    x2 = x.reshape(_BATCH, _ROWS * _D)
    mesh = plsc.VectorSubcoreMesh(core_axis_name="c", subcore_axis_name="s")
    run = functools.partial(
        pl.kernel,
        out_type=jax.ShapeDtypeStruct((_BATCH, _ROWS * _D), jnp.float32),
        mesh=mesh,
        scratch_types=[
            pltpu.VMEM((2, _CHUNK), jnp.int32),
            pltpu.VMEM((2, _CHUNK, _D), jnp.float32),
            pltpu.VMEM((2, _BATCH * _CELEM), jnp.float32),
        ] + [pltpu.SemaphoreType.DMA] * 6,
    )(_body)
    out = run(x2, sel, pos2)
    return out.reshape(_BATCH, _ROWS, _D)


# speedup vs baseline: 2.2588x; 2.0450x over previous
import functools

import jax
import jax.numpy as jnp
from jax import lax
from jax.experimental import pallas as pl
from jax.experimental.pallas import tpu as pltpu
from jax.experimental.pallas import tpu_sc as plsc

_D = 1024
_ROWS = 4096
_BATCH = 4
_NW = 32
_ROWS_PER_W = _ROWS // _NW
_CHUNK = 8
_NCHUNK = _ROWS_PER_W // _CHUNK
_CELEM = _CHUNK * _D
_GRP = 16
_NGRP = _CELEM // (16 * _GRP)


def _body(x_hbm, sel_hbm, pos_hbm, out_hbm, idx_v, pe_v, xb_v, *sems):
    pes = sems[0:2]
    xl = sems[2:4]
    ss = sems[4:6]
    wid = lax.axis_index("s") * 2 + lax.axis_index("c")
    base = wid * _ROWS_PER_W
    n_rows = 2 * _CHUNK + (wid >> 5)

    def row0(c):
        return base + c * _CHUNK

    def start_pe(c):
        p = c % 2
        pltpu.sync_copy(sel_hbm.at[pl.ds(row0(c), _CHUNK)], idx_v.at[p])
        return pltpu.async_copy(pos_hbm.at[idx_v.at[p]], pe_v.at[p], pes[p])

    def start_loads(c):
        q = c % 2
        return [pltpu.async_copy(
            x_hbm.at[pl.ds(b * _ROWS + row0(c), _CHUNK), :],
            xb_v.at[q, b], xl[q])
            for b in range(_BATCH)]

    def start_stores(c):
        q = c % 2
        return [pltpu.async_copy(
            xb_v.at[q, b],
            out_hbm.at[pl.ds(b * _ROWS + row0(c), _CHUNK), :], ss[q])
            for b in range(_BATCH)]

    pe_h = {0: start_pe(0)}
    ld_h = {0: start_loads(0)}
    st_h = {}
    for c in range(_NCHUNK):
        p = c % 2
        q = c % 2
        pe_h[c].wait()
        if c + 1 < _NCHUNK:
            pe_h[c + 1] = start_pe(c + 1)
        for h in ld_h[c]:
            h.wait()

        def add_half(h, carry, q=q, p=p):
            r = h >> 1
            jb = (h & 1) * (_D // 2)
            jb = pl.multiple_of(jb, 16)
            for u in range(_D // 32):
                v = pe_v[p, r, pl.ds(jb + u * 16, 16)]
                for b in range(_BATCH):
                    plsc.addupdate(xb_v.at[q, b, r, pl.ds(jb + u * 16, 16)],
                                   v)
            return carry

        lax.fori_loop(0, n_rows, add_half, 0)
        st_h[c] = start_stores(c)
        if c + 1 < _NCHUNK:
            if c - 1 >= 0:
                for h in st_h[c - 1]:
                    h.wait()
            ld_h[c + 1] = start_loads(c + 1)
    for c in (_NCHUNK - 2, _NCHUNK - 1):
        for h in st_h[c]:
            h.wait()


def kernel(x, pos_table, hash_index):
    sel = hash_index[:64, :64].reshape(-1).astype(jnp.int32)
    pos2 = pos_table.reshape(pos_table.shape[1], _D)
    x2 = x.reshape(_BATCH * _ROWS, _D)
    mesh = plsc.VectorSubcoreMesh(core_axis_name="c", subcore_axis_name="s")
    run = functools.partial(
        pl.kernel,
        out_type=jax.ShapeDtypeStruct((_BATCH * _ROWS, _D), jnp.float32),
        mesh=mesh,
        scratch_types=[
            pltpu.VMEM((2, _CHUNK), jnp.int32),
            pltpu.VMEM((2, _CHUNK, _D), jnp.float32),
            pltpu.VMEM((2, _BATCH, _CHUNK, _D), jnp.float32),
        ] + [pltpu.SemaphoreType.DMA] * 6,
    )(_body)
    out = run(x2, sel, pos2)
    return out.reshape(_BATCH, _ROWS, _D)


# speedup vs baseline: 2.9011x; 1.2844x over previous
import functools

import jax
import jax.numpy as jnp
from jax import lax
from jax.experimental import pallas as pl
from jax.experimental.pallas import tpu as pltpu
from jax.experimental.pallas import tpu_sc as plsc

_D = 1024
_ROWS = 4096
_BATCH = 4
_NW = 32
_ROWS_PER_W = _ROWS // _NW
_CHUNK = 8
_NCHUNK = _ROWS_PER_W // _CHUNK
_CELEM = _CHUNK * _D
_GRP = 16
_NGRP = _CELEM // (16 * _GRP)


def _body(x_hbm, sel_hbm, pos_hbm, out_hbm, idx_v, pe_v, xb_v, *sems):
    pes = sems[0:2]
    xl = sems[2:5]
    ss = sems[5:8]
    wid = lax.axis_index("s") * 2 + lax.axis_index("c")
    base = wid * _ROWS_PER_W
    n_rows = 2 * _CHUNK + (wid >> 5)

    def row0(c):
        return base + c * _CHUNK

    def start_pe(c):
        p = c % 2
        pltpu.sync_copy(sel_hbm.at[pl.ds(row0(c), _CHUNK)], idx_v.at[p])
        return pltpu.async_copy(pos_hbm.at[idx_v.at[p]], pe_v.at[p], pes[p])

    def start_loads(c):
        q = c % 3
        return [pltpu.async_copy(
            x_hbm.at[pl.ds(b * _ROWS + row0(c), _CHUNK), :],
            xb_v.at[q, b], xl[q])
            for b in range(_BATCH)]

    def start_stores(c):
        q = c % 3
        return [pltpu.async_copy(
            xb_v.at[q, b],
            out_hbm.at[pl.ds(b * _ROWS + row0(c), _CHUNK), :], ss[q])
            for b in range(_BATCH)]

    pe_h = {0: start_pe(0)}
    ld_h = {0: start_loads(0), 1: start_loads(1)}
    st_h = {}
    for c in range(_NCHUNK):
        p = c % 2
        q = c % 3
        pe_h[c].wait()
        if c + 1 < _NCHUNK:
            pe_h[c + 1] = start_pe(c + 1)
        for h in ld_h[c]:
            h.wait()

        def add_half(h, carry, q=q, p=p):
            r = h >> 1
            jb = (h & 1) * (_D // 2)
            jb = pl.multiple_of(jb, 16)
            for u in range(_D // 32):
                v = pe_v[p, r, pl.ds(jb + u * 16, 16)]
                for b in range(_BATCH):
                    plsc.addupdate(xb_v.at[q, b, r, pl.ds(jb + u * 16, 16)],
                                   v)
            return carry

        lax.fori_loop(0, n_rows, add_half, 0)
        st_h[c] = start_stores(c)
        if c + 2 < _NCHUNK:
            if c - 1 >= 0:
                for h in st_h[c - 1]:
                    h.wait()
            ld_h[c + 2] = start_loads(c + 2)
    for c in (_NCHUNK - 3, _NCHUNK - 2, _NCHUNK - 1):
        for h in st_h[c]:
            h.wait()


def kernel(x, pos_table, hash_index):
    sel = hash_index[:64, :64].reshape(-1).astype(jnp.int32)
    pos2 = pos_table.reshape(pos_table.shape[1], _D)
    x2 = x.reshape(_BATCH * _ROWS, _D)
    mesh = plsc.VectorSubcoreMesh(core_axis_name="c", subcore_axis_name="s")
    run = functools.partial(
        pl.kernel,
        out_type=jax.ShapeDtypeStruct((_BATCH * _ROWS, _D), jnp.float32),
        mesh=mesh,
        scratch_types=[
            pltpu.VMEM((2, _CHUNK), jnp.int32),
            pltpu.VMEM((2, _CHUNK, _D), jnp.float32),
            pltpu.VMEM((3, _BATCH, _CHUNK, _D), jnp.float32),
        ] + [pltpu.SemaphoreType.DMA] * 8,
    )(_body)
    out = run(x2, sel, pos2)
    return out.reshape(_BATCH, _ROWS, _D)
